# baseline (speedup 1.0000x reference)
import functools

import jax
import jax.numpy as jnp
from jax import lax
from jax.experimental import pallas as pl
from jax.experimental.pallas import tpu as pltpu
from jax.experimental.pallas import tpu_sc as plsc

_N_SPLITS = 16384
_L = 256
_N_CODE = 8192

_M_BLK = 512
_PREC = lax.Precision.DEFAULT

_NC = 2
_NS = 16
_NW = _NC * _NS
_B_PER_W = _N_SPLITS // _NW
_G_STEP = 64
_G_STEPS = _B_PER_W // _G_STEP


def _prep_block(b_ref, b2_ref, bbf_ref):
    b = b_ref[...]
    b2_ref[...] = jnp.sum(b * b, axis=1)[None, :]
    bbf_ref[...] = b.astype(jnp.bfloat16)


def _codebook_prep(q_codebook):
    return pl.pallas_call(
        _prep_block,
        out_shape=(jax.ShapeDtypeStruct((1, _N_CODE), jnp.float32),
                   jax.ShapeDtypeStruct((_N_CODE, _L), jnp.bfloat16)),
    )(q_codebook)


def _dist_argmin_block(a_ref, b_ref, b2_ref, idx_ref):
    am2 = a_ref[...] * (-2.0)
    b = b_ref[...]
    ab2 = lax.dot_general(
        am2, b, (((1,), (1,)), ((), ())),
        precision=_PREC, preferred_element_type=jnp.float32)
    a = am2 * (-0.5)
    a2 = jnp.sum(a * a, axis=1, keepdims=True)
    sq = (a2 + ab2) + b2_ref[...]
    min_sq = jnp.min(sq, axis=1, keepdims=True)

    ms_t = min_sq.T
    dmin = jnp.sqrt(jnp.maximum(ms_t, 0.0))
    cbits = lax.bitcast_convert_type(dmin * dmin, jnp.int32)
    t_t = jnp.full_like(dmin, -jnp.inf)
    for k in range(-4, 5):
        cand = lax.bitcast_convert_type(cbits + k, jnp.float32)
        ok = jnp.sqrt(jnp.maximum(cand, 0.0)) == dmin
        t_t = jnp.where(ok, cand, t_t)
    t = t_t.T

    ii = lax.broadcasted_iota(jnp.int32, sq.shape, 1)
    fi = lax.bitcast_convert_type(ii + jnp.int32(0x4B000000), jnp.float32)
    idx_f = jnp.min(jnp.where(sq <= t, fi, jnp.inf), axis=1, keepdims=True)
    idx = lax.bitcast_convert_type(idx_f, jnp.int32) - jnp.int32(0x4B000000)
    idx_ref[0, 0, :] = idx[:, 0]


def _nearest_indices(splits_flat, q_bf, b2):
    num_m = _N_SPLITS // _M_BLK
    out = pl.pallas_call(
        _dist_argmin_block,
        grid=(num_m,),
        in_specs=[
            pl.BlockSpec((_M_BLK, _L), lambda m: (m, 0)),
            pl.BlockSpec((_N_CODE, _L), lambda m: (0, 0)),
            pl.BlockSpec((1, _N_CODE), lambda m: (0, 0)),
        ],
        out_specs=pl.BlockSpec((1, 1, _M_BLK), lambda m: (m, 0, 0)),
        out_shape=jax.ShapeDtypeStruct((num_m, 1, _M_BLK), jnp.int32),
    )(splits_flat, q_bf, b2)
    return out.reshape(_N_SPLITS)


def _gather_codewords(q_codebook, c_syn, indices):
    mesh = plsc.VectorSubcoreMesh(core_axis_name="c", subcore_axis_name="s")

    @functools.partial(
        pl.kernel,
        out_type=(jax.ShapeDtypeStruct((_N_SPLITS, _L), jnp.float32),
                  jax.ShapeDtypeStruct((_N_SPLITS, _L), jnp.float32)),
        mesh=mesh,
        scratch_types=[
            pltpu.VMEM((_B_PER_W,), jnp.int32),
            pltpu.VMEM((_G_STEP, _L), jnp.float32),
            pltpu.VMEM((_G_STEP, _L), jnp.float32),
            pltpu.VMEM((_G_STEP, _L), jnp.float32),
            pltpu.VMEM((_G_STEP, _L), jnp.float32),
            pltpu.SemaphoreType.DMA,
            pltpu.SemaphoreType.DMA,
            pltpu.SemaphoreType.DMA,
            pltpu.SemaphoreType.DMA,
        ],
    )
    def k(tq_hbm, tc_hbm, idx_hbm, oq_hbm, oc_hbm,
          idx_v, rq0, rc0, rq1, rc1, smq0, smc0, smq1, smc1):
        wid = lax.axis_index("s") * _NC + lax.axis_index("c")
        base = wid * _B_PER_W
        pltpu.sync_copy(idx_hbm.at[pl.ds(base, _B_PER_W)], idx_v)

        bufs = ((rq0, rc0, smq0, smc0), (rq1, rc1, smq1, smc1))

        def issue(g, buf):
            rq, rc, smq, smc = buf
            cq = pltpu.async_copy(
                tq_hbm.at[idx_v.at[pl.ds(g * _G_STEP, _G_STEP)]], rq, smq)
            cc = pltpu.async_copy(
                tc_hbm.at[idx_v.at[pl.ds(g * _G_STEP, _G_STEP)]], rc, smc)
            return cq, cc

        pend = issue(0, bufs[0])
        for g in range(_G_STEPS):
            nxt = issue(g + 1, bufs[(g + 1) % 2]) if g + 1 < _G_STEPS else None
            rq, rc, _, _ = bufs[g % 2]
            cq, cc = pend
            cq.wait()
            pltpu.sync_copy(rq, oq_hbm.at[pl.ds(base + g * _G_STEP, _G_STEP)])
            cc.wait()
            pltpu.sync_copy(rc, oc_hbm.at[pl.ds(base + g * _G_STEP, _G_STEP)])
            pend = nxt

    return k(q_codebook, c_syn, indices)


def kernel(splits_flat, Q_codebook, C_syn):
    b2, q_bf = _codebook_prep(Q_codebook)
    indices = _nearest_indices(splits_flat, q_bf, b2)
    quant_words, ura_words = _gather_codewords(Q_codebook, C_syn, indices)
    return (indices, quant_words, ura_words)


# speedup vs baseline: 1.0580x; 1.0580x over previous
import functools

import jax
import jax.numpy as jnp
from jax import lax
from jax.experimental import pallas as pl
from jax.experimental.pallas import tpu as pltpu
from jax.experimental.pallas import tpu_sc as plsc

_N_SPLITS = 16384
_L = 256
_N_CODE = 8192

_M_BLK = 1024
_PREC = lax.Precision.DEFAULT

_NC = 2
_NS = 16
_NW = _NC * _NS
_B_PER_W = _N_SPLITS // _NW
_G_STEP = 128


def _prep_block(b_ref, b2_ref, bbf_ref):
    b = b_ref[...]
    b2_ref[...] = jnp.sum(b * b, axis=1)[None, :]
    bbf_ref[...] = b.astype(jnp.bfloat16)


def _codebook_prep(q_codebook):
    return pl.pallas_call(
        _prep_block,
        out_shape=(jax.ShapeDtypeStruct((1, _N_CODE), jnp.float32),
                   jax.ShapeDtypeStruct((_N_CODE, _L), jnp.bfloat16)),
    )(q_codebook)


def _dist_argmin_block(a_ref, b_ref, b2_ref, idx_ref):
    am2 = a_ref[...] * (-2.0)
    b = b_ref[...]
    ab2 = lax.dot_general(
        am2, b, (((1,), (1,)), ((), ())),
        precision=_PREC, preferred_element_type=jnp.float32)
    a = am2 * (-0.5)
    a2 = jnp.sum(a * a, axis=1, keepdims=True)
    sq = (a2 + ab2) + b2_ref[...]
    min_sq = jnp.min(sq, axis=1, keepdims=True)

    ms_t = min_sq.T
    dmin = jnp.sqrt(jnp.maximum(ms_t, 0.0))
    cbits = lax.bitcast_convert_type(dmin * dmin, jnp.int32)
    t_t = jnp.full_like(dmin, -jnp.inf)
    for k in range(-4, 5):
        cand = lax.bitcast_convert_type(cbits + k, jnp.float32)
        ok = jnp.sqrt(jnp.maximum(cand, 0.0)) == dmin
        t_t = jnp.where(ok, cand, t_t)
    t = t_t.T

    ii = lax.broadcasted_iota(jnp.int32, sq.shape, 1)
    fi = lax.bitcast_convert_type(ii + jnp.int32(0x4B000000), jnp.float32)
    idx_f = jnp.min(jnp.where(sq <= t, fi, jnp.inf), axis=1, keepdims=True)
    idx = lax.bitcast_convert_type(idx_f, jnp.int32) - jnp.int32(0x4B000000)
    idx_ref[0, 0, :] = idx[:, 0]


def _nearest_indices(splits_flat, q_bf, b2):
    num_m = _N_SPLITS // _M_BLK
    out = pl.pallas_call(
        _dist_argmin_block,
        grid=(num_m,),
        in_specs=[
            pl.BlockSpec((_M_BLK, _L), lambda m: (m, 0)),
            pl.BlockSpec((_N_CODE, _L), lambda m: (0, 0)),
            pl.BlockSpec((1, _N_CODE), lambda m: (0, 0)),
        ],
        out_specs=pl.BlockSpec((1, 1, _M_BLK), lambda m: (m, 0, 0)),
        out_shape=jax.ShapeDtypeStruct((num_m, 1, _M_BLK), jnp.int32),
    )(splits_flat, q_bf, b2)
    return out.reshape(_N_SPLITS)


def _gather_codewords(q_codebook, c_syn, indices):
    mesh = plsc.VectorSubcoreMesh(core_axis_name="c", subcore_axis_name="s")

    @functools.partial(
        pl.kernel,
        out_type=(jax.ShapeDtypeStruct((_N_SPLITS, _L), jnp.float32),
                  jax.ShapeDtypeStruct((_N_SPLITS, _L), jnp.float32)),
        mesh=mesh,
        scratch_types=[
            pltpu.VMEM((_B_PER_W,), jnp.int32),
            pltpu.VMEM((_G_STEP, _L), jnp.float32),
            pltpu.VMEM((_G_STEP, _L), jnp.float32),
            pltpu.SemaphoreType.DMA,
            pltpu.SemaphoreType.DMA,
        ],
    )
    def k(tq_hbm, tc_hbm, idx_hbm, oq_hbm, oc_hbm,
          idx_v, rq_v, rc_v, semq, semc):
        wid = lax.axis_index("s") * _NC + lax.axis_index("c")
        base = wid * _B_PER_W
        pltpu.sync_copy(idx_hbm.at[pl.ds(base, _B_PER_W)], idx_v)

        @pl.loop(0, _B_PER_W, step=_G_STEP)
        def _(g):
            cq = pltpu.async_copy(
                tq_hbm.at[idx_v.at[pl.ds(g, _G_STEP)]], rq_v, semq)
            cc = pltpu.async_copy(
                tc_hbm.at[idx_v.at[pl.ds(g, _G_STEP)]], rc_v, semc)
            cq.wait()
            pltpu.sync_copy(rq_v, oq_hbm.at[pl.ds(base + g, _G_STEP)])
            cc.wait()
            pltpu.sync_copy(rc_v, oc_hbm.at[pl.ds(base + g, _G_STEP)])

    return k(q_codebook, c_syn, indices)


def kernel(splits_flat, Q_codebook, C_syn):
    b2, q_bf = _codebook_prep(Q_codebook)
    indices = _nearest_indices(splits_flat, q_bf, b2)
    quant_words, ura_words = _gather_codewords(Q_codebook, C_syn, indices)
    return (indices, quant_words, ura_words)


# speedup vs baseline: 1.0692x; 1.0106x over previous
import functools

import jax
import jax.numpy as jnp
from jax import lax
from jax.experimental import pallas as pl
from jax.experimental.pallas import tpu as pltpu
from jax.experimental.pallas import tpu_sc as plsc

_N_SPLITS = 16384
_L = 256
_N_CODE = 8192

_M_BLK = 1024
_PREC = lax.Precision.DEFAULT

_NC = 2
_NS = 16
_NW = _NC * _NS
_B_PER_W = _N_SPLITS // _NW
_G_STEP = 128


def _dist_argmin_block(a_ref, b_ref, idx_ref, b2_s, bbf_s):
    am2 = a_ref[...] * (-2.0)

    @pl.when(pl.program_id(0) == 0)
    def _():
        b0 = b_ref[...]
        b2_s[...] = jnp.sum(b0 * b0, axis=1)[None, :]
        bbf_s[...] = b0.astype(jnp.bfloat16)

    b = bbf_s[...]
    ab2 = lax.dot_general(
        am2, b, (((1,), (1,)), ((), ())),
        precision=_PREC, preferred_element_type=jnp.float32)
    a = am2 * (-0.5)
    a2 = jnp.sum(a * a, axis=1, keepdims=True)
    sq = (a2 + ab2) + b2_s[...]
    min_sq = jnp.min(sq, axis=1, keepdims=True)

    ms_t = min_sq.T
    dmin = jnp.sqrt(jnp.maximum(ms_t, 0.0))
    cbits = lax.bitcast_convert_type(dmin * dmin, jnp.int32)
    t_t = jnp.full_like(dmin, -jnp.inf)
    for k in range(-4, 5):
        cand = lax.bitcast_convert_type(cbits + k, jnp.float32)
        ok = jnp.sqrt(jnp.maximum(cand, 0.0)) == dmin
        t_t = jnp.where(ok, cand, t_t)
    t = t_t.T

    ii = lax.broadcasted_iota(jnp.int32, sq.shape, 1)
    fi = lax.bitcast_convert_type(ii + jnp.int32(0x4B000000), jnp.float32)
    idx_f = jnp.min(jnp.where(sq <= t, fi, jnp.inf), axis=1, keepdims=True)
    idx = lax.bitcast_convert_type(idx_f, jnp.int32) - jnp.int32(0x4B000000)
    idx_ref[0, 0, :] = idx[:, 0]


def _nearest_indices(splits_flat, q_codebook):
    num_m = _N_SPLITS // _M_BLK
    out = pl.pallas_call(
        _dist_argmin_block,
        grid=(num_m,),
        in_specs=[
            pl.BlockSpec((_M_BLK, _L), lambda m: (m, 0)),
            pl.BlockSpec((_N_CODE, _L), lambda m: (0, 0)),
        ],
        out_specs=pl.BlockSpec((1, 1, _M_BLK), lambda m: (m, 0, 0)),
        out_shape=jax.ShapeDtypeStruct((num_m, 1, _M_BLK), jnp.int32),
        scratch_shapes=[
            pltpu.VMEM((1, _N_CODE), jnp.float32),
            pltpu.VMEM((_N_CODE, _L), jnp.bfloat16),
        ],
    )(splits_flat, q_codebook)
    return out.reshape(_N_SPLITS)


def _gather_codewords(q_codebook, c_syn, indices):
    mesh = plsc.VectorSubcoreMesh(core_axis_name="c", subcore_axis_name="s")

    @functools.partial(
        pl.kernel,
        out_type=(jax.ShapeDtypeStruct((_N_SPLITS, _L), jnp.float32),
                  jax.ShapeDtypeStruct((_N_SPLITS, _L), jnp.float32)),
        mesh=mesh,
        scratch_types=[
            pltpu.VMEM((_B_PER_W,), jnp.int32),
            pltpu.VMEM((_G_STEP, _L), jnp.float32),
            pltpu.VMEM((_G_STEP, _L), jnp.float32),
            pltpu.SemaphoreType.DMA,
            pltpu.SemaphoreType.DMA,
        ],
    )
    def k(tq_hbm, tc_hbm, idx_hbm, oq_hbm, oc_hbm,
          idx_v, rq_v, rc_v, semq, semc):
        wid = lax.axis_index("s") * _NC + lax.axis_index("c")
        base = wid * _B_PER_W
        pltpu.sync_copy(idx_hbm.at[pl.ds(base, _B_PER_W)], idx_v)

        @pl.loop(0, _B_PER_W, step=_G_STEP)
        def _(g):
            cq = pltpu.async_copy(
                tq_hbm.at[idx_v.at[pl.ds(g, _G_STEP)]], rq_v, semq)
            cc = pltpu.async_copy(
                tc_hbm.at[idx_v.at[pl.ds(g, _G_STEP)]], rc_v, semc)
            cq.wait()
            pltpu.sync_copy(rq_v, oq_hbm.at[pl.ds(base + g, _G_STEP)])
            cc.wait()
            pltpu.sync_copy(rc_v, oc_hbm.at[pl.ds(base + g, _G_STEP)])

    return k(q_codebook, c_syn, indices)


def kernel(splits_flat, Q_codebook, C_syn):
    indices = _nearest_indices(splits_flat, Q_codebook)
    quant_words, ura_words = _gather_codewords(Q_codebook, C_syn, indices)
    return (indices, quant_words, ura_words)
